# Initial kernel scaffold; baseline (speedup 1.0000x reference)
import jax
import jax.numpy as jnp
from jax.experimental import pallas as pl


def kernel(obs_encoding, lane_encoding, same_obs_mask, W, b):
    raise NotImplementedError("write your pallas kernel here")


# speedup vs baseline: 1.2584x; 1.2584x over previous
import functools

import jax
import jax.numpy as jnp
from jax import lax
from jax.experimental import pallas as pl
from jax.experimental.pallas import tpu as pltpu
from jax.experimental.pallas import tpu_sc as plsc

M = 320000
D = 128
O = 512
N = 10000

BM = 2000


def _encode_body(lane_ref, w_ref, b_ref, out_ref):
    acc = jnp.dot(lane_ref[...], w_ref[...], preferred_element_type=jnp.float32)
    out_ref[...] = jnp.maximum(acc + b_ref[...], 0.0)


def _encode(lane, w_t, b2):
    return pl.pallas_call(
        _encode_body,
        grid=(M // BM,),
        in_specs=[
            pl.BlockSpec((BM, D), lambda i: (i, 0)),
            pl.BlockSpec((D, O), lambda i: (0, 0)),
            pl.BlockSpec((1, O), lambda i: (0, 0)),
        ],
        out_specs=pl.BlockSpec((BM, O), lambda i: (i, 0)),
        out_shape=jax.ShapeDtypeStruct((M, O), jnp.float32),
    )(lane, w_t, b2)


def _zero_body(out_ref):
    out_ref[...] = jnp.zeros_like(out_ref)


def _zeros():
    return pl.pallas_call(
        _zero_body,
        grid=(10,),
        out_specs=pl.BlockSpec((N // 10, 2 * O), lambda i: (i, 0)),
        out_shape=jax.ShapeDtypeStruct((N, 2 * O), jnp.float32),
    )()


NROW = 16
NCOL = 2
CH = M // NROW
RT = 160
NT = M // RT
CW = O // NCOL
KC = CW // 16
WIN = 8

_PSID, _DONE, _SBWIN, _DIRTY = 0, 1, 2, 3


def _seg_body(enc_hbm, seg_hbm, out_hbm, seg_t, enc_t, stg_max, stg_sum,
              acc_v, st_i, st_f, boot):
    cw = lax.axis_index("c")
    rw = lax.axis_index("s")
    r0 = rw * CH
    r1 = r0 + CH
    col0 = cw * CW

    @pl.when(rw > 0)
    def _():
        pltpu.sync_copy(seg_hbm.at[pl.ds(pl.multiple_of(r0 - 8, 8), 8)],
                        boot.at[pl.ds(0, 8)])

    pltpu.sync_copy(seg_hbm.at[pl.ds(pl.multiple_of(r1 - 8, 8), 8)],
                    boot.at[pl.ds(8, 8)])
    bvec = boot[...]
    b_lo = jnp.where(rw > 0, ((bvec[7] + 8) >> 3) << 3, jnp.int32(0))
    b_hi = jnp.where(rw == NROW - 1, jnp.int32(N), ((bvec[15] + 8) >> 3) << 3)

    zero16 = jnp.zeros((16,), jnp.float32)

    def _zero_stage():
        def zrow(s, _):
            for k in range(KC):
                stg_max[s, pl.ds(k * 16, 16)] = zero16
                stg_sum[s, pl.ds(k * 16, 16)] = zero16
            return 0
        lax.fori_loop(0, WIN, zrow, 0)

    def _flush(row_base):
        rb = pl.multiple_of(row_base, 8)
        pltpu.sync_copy(stg_max, out_hbm.at[pl.ds(rb, WIN), pl.ds(col0, CW)])
        pltpu.sync_copy(stg_sum,
                        out_hbm.at[pl.ds(rb, WIN), pl.ds(O + col0, CW)])
        _zero_stage()

    _zero_stage()
    st_i[_PSID] = jnp.int32(-1)
    st_i[_DONE] = (b_lo >= b_hi).astype(jnp.int32)
    st_i[_SBWIN] = b_lo
    st_i[_DIRTY] = jnp.int32(0)
    st_f[0] = jnp.float32(0.0)

    def tile(t, _):
        start = pl.multiple_of(r0 + t * RT, 8)
        ok = jnp.logical_and(start < M, st_i[_DONE] == 0)

        @pl.when(ok)
        def _():
            pltpu.sync_copy(seg_hbm.at[pl.ds(start, RT)], seg_t)
            pltpu.sync_copy(enc_hbm.at[pl.ds(start, RT), pl.ds(col0, CW)],
                            enc_t)
            amax0 = tuple(acc_v[0, pl.ds(k * 16, 16)] for k in range(KC))
            asum0 = tuple(acc_v[1, pl.ds(k * 16, 16)] for k in range(KC))
            init = (st_i[_PSID], st_f[0], st_i[_DONE] != 0, st_i[_SBWIN],
                    st_i[_DIRTY] != 0, amax0, asum0)

            def group(g, rc):
                sv = seg_t[pl.ds(pl.multiple_of(g * 16, 16), 16)]
                for jj in range(16):
                    psid, cnt, done, sb_win, dirty, amax, asum = rc
                    sid = sv[jj]
                    j = g * 16 + jj
                    active = jnp.logical_and(jnp.logical_not(done),
                                             sid < b_hi)
                    store_ok = jnp.logical_and(active, sid >= b_lo)
                    new_run = sid != psid

                    emit = jnp.logical_and(
                        new_run,
                        jnp.logical_and(psid >= b_lo, psid < b_hi))

                    @pl.when(emit)
                    def _(psid=psid, cnt=cnt, sb_win=sb_win, amax=amax,
                          asum=asum):
                        slot = psid - sb_win
                        d = jnp.maximum(cnt, jnp.float32(1.0))
                        for k in range(KC):
                            stg_max[slot, pl.ds(k * 16, 16)] = amax[k]
                            stg_sum[slot, pl.ds(k * 16, 16)] = asum[k] / d

                    dirty2 = jnp.logical_or(dirty, emit)

                    win_sid = (sid >> 3) << 3
                    jump = jnp.logical_and(store_ok, win_sid != sb_win)

                    @pl.when(jnp.logical_and(jump, dirty2))
                    def _(sb_win=sb_win):
                        _flush(sb_win)

                    sb_win2 = jnp.where(jump, win_sid, sb_win)
                    dirty3 = jnp.logical_and(dirty2, jnp.logical_not(jump))

                    vs = [enc_t[j, pl.ds(k * 16, 16)] for k in range(KC)]
                    namax = tuple(
                        jnp.where(new_run, vs[k],
                                  jnp.maximum(amax[k], vs[k]))
                        for k in range(KC))
                    nasum = tuple(
                        jnp.where(new_run, vs[k], asum[k] + vs[k])
                        for k in range(KC))
                    ncnt = jnp.where(new_run, jnp.float32(1.0), cnt + 1.0)
                    ndone = jnp.logical_or(done, sid >= b_hi)
                    rc = (sid, ncnt, ndone, sb_win2, dirty3, namax, nasum)
                return rc

            psid, cnt, done, sb_win, dirty, amax, asum = lax.fori_loop(
                0, RT // 16, group, init)
            for k in range(KC):
                acc_v[0, pl.ds(k * 16, 16)] = amax[k]
                acc_v[1, pl.ds(k * 16, 16)] = asum[k]
            st_i[_PSID] = psid
            st_i[_DONE] = done.astype(jnp.int32)
            st_i[_SBWIN] = sb_win
            st_i[_DIRTY] = dirty.astype(jnp.int32)
            st_f[0] = cnt
        return 0

    lax.fori_loop(0, NT, tile, 0)

    psid = st_i[_PSID]
    emit_f = jnp.logical_and(psid >= b_lo, psid < b_hi)

    @pl.when(emit_f)
    def _():
        slot = psid - st_i[_SBWIN]
        d = jnp.maximum(st_f[0], jnp.float32(1.0))
        for k in range(KC):
            stg_max[slot, pl.ds(k * 16, 16)] = acc_v[0, pl.ds(k * 16, 16)]
            stg_sum[slot, pl.ds(k * 16, 16)] = acc_v[1, pl.ds(k * 16, 16)] / d

    @pl.when(jnp.logical_or(st_i[_DIRTY] != 0, emit_f))
    def _():
        _flush(st_i[_SBWIN])


def _segment_pool(enc, seg, out_ref):
    mesh = plsc.VectorSubcoreMesh(core_axis_name="c", subcore_axis_name="s")
    f = functools.partial(
        pl.kernel,
        mesh=mesh,
        out_type=(),
        scratch_types=[
            pltpu.VMEM((RT,), jnp.int32),
            pltpu.VMEM((RT, CW), jnp.float32),
            pltpu.VMEM((WIN, CW), jnp.float32),
            pltpu.VMEM((WIN, CW), jnp.float32),
            pltpu.VMEM((2, CW), jnp.float32),
            pltpu.SMEM((8,), jnp.int32),
            pltpu.SMEM((8,), jnp.float32),
            pltpu.VMEM((16,), jnp.int32),
        ],
    )(_seg_body)
    f(enc, seg, out_ref)


def kernel(obs_encoding, lane_encoding, same_obs_mask, W, b):
    del obs_encoding
    w_t = W.T
    b2 = b.reshape(1, O)
    enc = _encode(lane_encoding, w_t, b2)
    seg = same_obs_mask.reshape(M)
    out_ref = jax.new_ref(_zeros())
    _segment_pool(enc, seg, out_ref)
    return jax.freeze(out_ref)


# speedup vs baseline: 1.4232x; 1.1310x over previous
import functools

import jax
import jax.numpy as jnp
from jax import lax
from jax.experimental import pallas as pl
from jax.experimental.pallas import tpu as pltpu
from jax.experimental.pallas import tpu_sc as plsc

M = 320000
D = 128
O = 512
N = 10000

BM = 2000


def _encode_body(lane_ref, w_ref, b_ref, out_ref):
    acc = jnp.dot(lane_ref[...], w_ref[...], preferred_element_type=jnp.float32)
    out_ref[...] = jnp.maximum(acc + b_ref[...], 0.0)


def _encode(lane, w_t, b2):
    return pl.pallas_call(
        _encode_body,
        grid=(M // BM,),
        in_specs=[
            pl.BlockSpec((BM, D), lambda i: (i, 0)),
            pl.BlockSpec((D, O), lambda i: (0, 0)),
            pl.BlockSpec((1, O), lambda i: (0, 0)),
        ],
        out_specs=pl.BlockSpec((BM, O), lambda i: (i, 0)),
        out_shape=jax.ShapeDtypeStruct((M, O), jnp.float32),
    )(lane, w_t, b2)


def _zero_body(out_ref):
    out_ref[...] = jnp.zeros_like(out_ref)


def _zeros():
    return pl.pallas_call(
        _zero_body,
        grid=(10,),
        out_specs=pl.BlockSpec((N // 10, 2 * O), lambda i: (i, 0)),
        out_shape=jax.ShapeDtypeStruct((N, 2 * O), jnp.float32),
    )()


NROW = 16
NCOL = 2
CH = M // NROW
RT = 160
NT = M // RT
CW = O // NCOL
KC = CW // 16
WIN = 8

_PSID, _DONE, _SBWIN, _DIRTY, _ISS0, _ISS1 = 0, 1, 2, 3, 4, 5


def _seg_body(enc_hbm, seg_hbm, out_hbm, seg2, enc2,
              stg_max, stg_sum, acc_v, st_i, st_f, boot, sem_a, sem_b):
    cw = lax.axis_index("c")
    rw = lax.axis_index("s")
    r0 = rw * CH
    r1 = r0 + CH
    col0 = cw * CW

    @pl.when(rw > 0)
    def _():
        pltpu.sync_copy(seg_hbm.at[pl.ds(pl.multiple_of(r0 - 8, 8), 8)],
                        boot.at[pl.ds(0, 8)])

    pltpu.sync_copy(seg_hbm.at[pl.ds(pl.multiple_of(r1 - 8, 8), 8)],
                    boot.at[pl.ds(8, 8)])
    bvec = boot[...]
    b_lo = jnp.where(rw > 0, ((bvec[7] + 8) >> 3) << 3, jnp.int32(0))
    b_hi = jnp.where(rw == NROW - 1, jnp.int32(N), ((bvec[15] + 8) >> 3) << 3)

    zero16 = jnp.zeros((16,), jnp.float32)

    def _zero_stage():
        def zrow(s, _):
            for k in range(KC):
                stg_max[s, pl.ds(k * 16, 16)] = zero16
                stg_sum[s, pl.ds(k * 16, 16)] = zero16
            return 0
        lax.fori_loop(0, WIN, zrow, 0)

    def _flush(row_base):
        rb = pl.multiple_of(row_base, 8)
        pltpu.sync_copy(stg_max, out_hbm.at[pl.ds(rb, WIN), pl.ds(col0, CW)])
        pltpu.sync_copy(stg_sum,
                        out_hbm.at[pl.ds(rb, WIN), pl.ds(O + col0, CW)])
        _zero_stage()

    _zero_stage()
    st_i[_PSID] = jnp.int32(-1)
    st_i[_DONE] = (b_lo >= b_hi).astype(jnp.int32)
    st_i[_SBWIN] = b_lo
    st_i[_DIRTY] = jnp.int32(0)
    st_i[_ISS0] = jnp.int32(0)
    st_i[_ISS1] = jnp.int32(0)
    st_f[0] = jnp.float32(0.0)

    def _issue_p(t_start, parity):
        ts = pl.multiple_of(t_start, 8)

        @pl.when(parity == 0)
        def _():
            pltpu.async_copy(seg_hbm.at[pl.ds(ts, RT)],
                             seg2.at[pl.ds(0, RT)], sem_a)
            pltpu.async_copy(enc_hbm.at[pl.ds(ts, RT), pl.ds(col0, CW)],
                             enc2.at[pl.ds(0, RT), :], sem_a)
            st_i[_ISS0] = jnp.int32(1)

        @pl.when(parity == 1)
        def _():
            pltpu.async_copy(seg_hbm.at[pl.ds(ts, RT)],
                             seg2.at[pl.ds(RT, RT)], sem_b)
            pltpu.async_copy(enc_hbm.at[pl.ds(ts, RT), pl.ds(col0, CW)],
                             enc2.at[pl.ds(RT, RT), :], sem_b)
            st_i[_ISS1] = jnp.int32(1)

    def _wait_p(parity):
        @pl.when(parity == 0)
        def _():
            pltpu.make_async_copy(seg_hbm.at[pl.ds(0, RT)],
                                  seg2.at[pl.ds(0, RT)], sem_a).wait()
            pltpu.make_async_copy(enc_hbm.at[pl.ds(0, RT), pl.ds(0, CW)],
                                  enc2.at[pl.ds(0, RT), :], sem_a).wait()
            st_i[_ISS0] = jnp.int32(0)

        @pl.when(parity == 1)
        def _():
            pltpu.make_async_copy(seg_hbm.at[pl.ds(0, RT)],
                                  seg2.at[pl.ds(RT, RT)], sem_b).wait()
            pltpu.make_async_copy(enc_hbm.at[pl.ds(0, RT), pl.ds(0, CW)],
                                  enc2.at[pl.ds(RT, RT), :], sem_b).wait()
            st_i[_ISS1] = jnp.int32(0)

    @pl.when(st_i[_DONE] == 0)
    def _():
        _issue_p(r0, jnp.int32(0))

    def tile(t, _):
        parity = jnp.bitwise_and(t, 1)
        start = pl.multiple_of(r0 + t * RT, 8)
        ok = jnp.logical_and(start < M, st_i[_DONE] == 0)

        @pl.when(ok)
        def _():
            _wait_p(parity)
            nstart = start + RT

            @pl.when(nstart < M)
            def _():
                _issue_p(nstart, 1 - parity)

            pb = parity * RT
            amax0 = tuple(acc_v[0, pl.ds(k * 16, 16)] for k in range(KC))
            asum0 = tuple(acc_v[1, pl.ds(k * 16, 16)] for k in range(KC))
            init = (st_i[_PSID], st_f[0], st_i[_DONE] != 0, st_i[_SBWIN],
                    st_i[_DIRTY] != 0, amax0, asum0)

            def group(g, rc):
                sv = seg2[pl.ds(pl.multiple_of(pb + g * 16, 16), 16)]
                for jj in range(16):
                    psid, cnt, done, sb_win, dirty, amax, asum = rc
                    sid = sv[jj]
                    j = g * 16 + jj
                    active = jnp.logical_and(jnp.logical_not(done),
                                             sid < b_hi)
                    store_ok = jnp.logical_and(active, sid >= b_lo)
                    new_run = sid != psid

                    emit = jnp.logical_and(
                        new_run,
                        jnp.logical_and(psid >= b_lo, psid < b_hi))

                    @pl.when(emit)
                    def _(psid=psid, cnt=cnt, sb_win=sb_win, amax=amax,
                          asum=asum):
                        slot = psid - sb_win
                        d = jnp.maximum(cnt, jnp.float32(1.0))
                        for k in range(KC):
                            stg_max[slot, pl.ds(k * 16, 16)] = amax[k]
                            stg_sum[slot, pl.ds(k * 16, 16)] = asum[k] / d

                    dirty2 = jnp.logical_or(dirty, emit)

                    win_sid = (sid >> 3) << 3
                    jump = jnp.logical_and(store_ok, win_sid != sb_win)

                    @pl.when(jnp.logical_and(jump, dirty2))
                    def _(sb_win=sb_win):
                        _flush(sb_win)

                    sb_win2 = jnp.where(jump, win_sid, sb_win)
                    dirty3 = jnp.logical_and(dirty2, jnp.logical_not(jump))

                    vs = [enc2[pb + j, pl.ds(k * 16, 16)] for k in range(KC)]
                    namax = tuple(
                        jnp.where(new_run, vs[k],
                                  jnp.maximum(amax[k], vs[k]))
                        for k in range(KC))
                    nasum = tuple(
                        jnp.where(new_run, vs[k], asum[k] + vs[k])
                        for k in range(KC))
                    ncnt = jnp.where(new_run, jnp.float32(1.0), cnt + 1.0)
                    ndone = jnp.logical_or(done, sid >= b_hi)
                    rc = (sid, ncnt, ndone, sb_win2, dirty3, namax, nasum)
                return rc

            psid, cnt, done, sb_win, dirty, amax, asum = lax.fori_loop(
                0, RT // 16, group, init)
            for k in range(KC):
                acc_v[0, pl.ds(k * 16, 16)] = amax[k]
                acc_v[1, pl.ds(k * 16, 16)] = asum[k]
            st_i[_PSID] = psid
            st_i[_DONE] = done.astype(jnp.int32)
            st_i[_SBWIN] = sb_win
            st_i[_DIRTY] = dirty.astype(jnp.int32)
            st_f[0] = cnt
        return 0

    lax.fori_loop(0, NT, tile, 0)

    @pl.when(st_i[_ISS0] == 1)
    def _():
        _wait_p(jnp.int32(0))

    @pl.when(st_i[_ISS1] == 1)
    def _():
        _wait_p(jnp.int32(1))

    psid = st_i[_PSID]
    emit_f = jnp.logical_and(psid >= b_lo, psid < b_hi)

    @pl.when(emit_f)
    def _():
        slot = psid - st_i[_SBWIN]
        d = jnp.maximum(st_f[0], jnp.float32(1.0))
        for k in range(KC):
            stg_max[slot, pl.ds(k * 16, 16)] = acc_v[0, pl.ds(k * 16, 16)]
            stg_sum[slot, pl.ds(k * 16, 16)] = acc_v[1, pl.ds(k * 16, 16)] / d

    @pl.when(jnp.logical_or(st_i[_DIRTY] != 0, emit_f))
    def _():
        _flush(st_i[_SBWIN])


def _segment_pool(enc, seg, out_ref):
    mesh = plsc.VectorSubcoreMesh(core_axis_name="c", subcore_axis_name="s")
    f = functools.partial(
        pl.kernel,
        mesh=mesh,
        out_type=(),
        scratch_types=[
            pltpu.VMEM((2 * RT,), jnp.int32),
            pltpu.VMEM((2 * RT, CW), jnp.float32),
            pltpu.VMEM((WIN, CW), jnp.float32),
            pltpu.VMEM((WIN, CW), jnp.float32),
            pltpu.VMEM((2, CW), jnp.float32),
            pltpu.SMEM((8,), jnp.int32),
            pltpu.SMEM((8,), jnp.float32),
            pltpu.VMEM((16,), jnp.int32),
            pltpu.SemaphoreType.DMA,
            pltpu.SemaphoreType.DMA,
        ],
    )(_seg_body)
    f(enc, seg, out_ref)


def kernel(obs_encoding, lane_encoding, same_obs_mask, W, b):
    del obs_encoding
    w_t = W.T
    b2 = b.reshape(1, O)
    enc = _encode(lane_encoding, w_t, b2)
    seg = same_obs_mask.reshape(M)
    out_ref = jax.new_ref(_zeros())
    _segment_pool(enc, seg, out_ref)
    return jax.freeze(out_ref)
